# baseline (speedup 1.0000x reference)
import functools

import jax
import jax.numpy as jnp
from jax import lax
from jax.experimental import pallas as pl
from jax.experimental.pallas import tpu as pltpu
from jax.experimental.pallas import tpu_sc as plsc

B, S, D, MAX_LEN = 4, 4096, 768, 8192
LANES = 16
NUM_WORKERS = 32
ROWS_PER_WORKER = (B * S) // NUM_WORKERS
CHUNK = 32
NCHUNKS = ROWS_PER_WORKER // CHUNK
VECS_PER_ROW = D // LANES


def _sc_kernel(x_hbm, len_hbm, pe_hbm, out_hbm, len_v, idx_v, xb, peb, sem):
    cid = lax.axis_index("c")
    sid = lax.axis_index("s")
    wid = sid * 2 + cid

    pltpu.sync_copy(len_hbm, len_v)
    lane = lax.iota(jnp.int32, 16)
    b = wid // (S // ROWS_PER_WORKER)
    len_vec = len_v[...]
    b_vec = jnp.full((16,), 0, jnp.int32) + b
    length_vec = lax.gather(
        len_vec,
        b_vec[:, None],
        lax.GatherDimensionNumbers(
            offset_dims=(), collapsed_slice_dims=(0,), start_index_map=(0,)),
        (1,),
        mode=lax.GatherScatterMode.PROMISE_IN_BOUNDS,
    )

    row_base = wid * ROWS_PER_WORKER
    s_base = row_base % S

    def chunk_body(c, _):
        row0 = row_base + c * CHUNK
        s0 = s_base + c * CHUNK

        pltpu.sync_copy(x_hbm.at[pl.ds(row0, CHUNK)], xb)

        for j in range(CHUNK // LANES):
            pos = length_vec - (s0 + j * LANES) - lane
            idx_v[pl.ds(j * LANES, LANES)] = jnp.maximum(pos, 0)
        pltpu.async_copy(pe_hbm.at[idx_v], peb, sem).wait()

        def row_body(r, _):
            for j in range(VECS_PER_ROW):
                sl = pl.ds(j * LANES, LANES)
                xb[r, sl] = xb[r, sl] + peb[r, sl]
            return 0

        lax.fori_loop(0, CHUNK, row_body, 0)

        pltpu.sync_copy(xb, out_hbm.at[pl.ds(row0, CHUNK)])
        return 0

    lax.fori_loop(0, NCHUNKS, chunk_body, 0)


def kernel(x, lengths, pe):
    n_batch, n_seq, d_emb = x.shape
    xf = x.reshape(n_batch * n_seq, d_emb)
    len_pad = jnp.zeros((16,), jnp.int32).at[:n_batch].set(lengths)

    mesh = plsc.VectorSubcoreMesh(core_axis_name="c", subcore_axis_name="s")
    run = functools.partial(
        pl.kernel,
        mesh=mesh,
        out_type=jax.ShapeDtypeStruct((n_batch * n_seq, d_emb), jnp.float32),
        scratch_types=[
            pltpu.VMEM((16,), jnp.int32),
            pltpu.VMEM((CHUNK,), jnp.int32),
            pltpu.VMEM((CHUNK, D), jnp.float32),
            pltpu.VMEM((CHUNK, D), jnp.float32),
            pltpu.SemaphoreType.DMA,
        ],
    )(_sc_kernel)
    out = run(xf, len_pad, pe)
    return out.reshape(n_batch, n_seq, d_emb)
